# Initial kernel scaffold; baseline (speedup 1.0000x reference)
import jax
import jax.numpy as jnp
from jax.experimental import pallas as pl


def kernel(x, edge_index, edge_weight, W):
    raise NotImplementedError("write your pallas kernel here")


# speedup vs baseline: 3.5338x; 3.5338x over previous
import functools

import jax
import jax.numpy as jnp
from jax import lax
from jax.experimental import pallas as pl
from jax.experimental.pallas import tpu as pltpu
from jax.experimental.pallas import tpu_sc as plsc

NC = 2
NS = 16
NW = NC * NS
LANES = 16
CHUNK = 128


def _sc_aggregate(x, src, dst, ew, n_nodes, d, k_chunks):
    zeros = jnp.zeros((n_nodes, d), jnp.float32)
    rows_per_sub = n_nodes // NS

    mesh = plsc.VectorSubcoreMesh(core_axis_name="c", subcore_axis_name="s")

    @functools.partial(
        pl.kernel,
        out_type=jax.ShapeDtypeStruct((NC, n_nodes, d), jnp.float32),
        mesh=mesh,
        scratch_types=[
            pltpu.VMEM((CHUNK,), jnp.int32),
            pltpu.VMEM((CHUNK,), jnp.int32),
            pltpu.VMEM((CHUNK,), jnp.float32),
            pltpu.VMEM((CHUNK, d), jnp.float32),
            pltpu.VMEM_SHARED((n_nodes, d), jnp.float32),
            pltpu.SemaphoreType.DMA,
        ],
    )
    def agg_kernel(x_hbm, src_hbm, dst_hbm, ew_hbm, z_hbm, part_hbm,
                   src_v, dst_v, w_v, rows_v, acc_sh, sem):
        c = lax.axis_index("c")
        s = lax.axis_index("s")
        wid = s * NC + c

        pltpu.sync_copy(z_hbm.at[pl.ds(s * rows_per_sub, rows_per_sub)],
                        acc_sh.at[pl.ds(s * rows_per_sub, rows_per_sub)])
        plsc.subcore_barrier()

        def chunk_body(k, carry):
            off = (wid * k_chunks + k) * CHUNK
            pltpu.sync_copy(src_hbm.at[pl.ds(off, CHUNK)], src_v)
            pltpu.sync_copy(dst_hbm.at[pl.ds(off, CHUNK)], dst_v)
            pltpu.sync_copy(ew_hbm.at[pl.ds(off, CHUNK)], w_v)
            pltpu.async_copy(x_hbm.at[src_v], rows_v, sem).wait()

            def scale_group(g, cc):
                wv = w_v[pl.ds(g * LANES, LANES)]
                for l in range(LANES):
                    e = g * LANES + l
                    wsc = wv[l]
                    for j in range(d // LANES):
                        sl = pl.ds(j * LANES, LANES)
                        rows_v[e, sl] = rows_v[e, sl] * wsc
                return cc

            lax.fori_loop(0, CHUNK // LANES, scale_group, 0)
            pltpu.sync_copy(rows_v, acc_sh.at[dst_v], add=True)
            return carry

        lax.fori_loop(0, k_chunks, chunk_body, 0)

        plsc.subcore_barrier()
        pltpu.sync_copy(acc_sh.at[pl.ds(s * rows_per_sub, rows_per_sub)],
                        part_hbm.at[c, pl.ds(s * rows_per_sub, rows_per_sub)])

    return agg_kernel(x, src, dst, ew, zeros)


def _tc_project(part, w, n_nodes, d, d_out):
    block_rows = next(b for b in (1024, 512, 256, 128, 8, 1) if n_nodes % b == 0)
    def body(p_ref, w_ref, o_ref):
        a = p_ref[0] + p_ref[1]
        y = jnp.dot(a, w_ref[...], preferred_element_type=jnp.float32)
        o_ref[...] = jnp.maximum(y, 0.0)

    return pl.pallas_call(
        body,
        grid=(n_nodes // block_rows,),
        in_specs=[
            pl.BlockSpec((NC, block_rows, d), lambda i: (0, i, 0)),
            pl.BlockSpec((d, d_out), lambda i: (0, 0)),
        ],
        out_specs=pl.BlockSpec((block_rows, d_out), lambda i: (i, 0)),
        out_shape=jax.ShapeDtypeStruct((n_nodes, d_out), jnp.float32),
    )(part, w)


def kernel(x, edge_index, edge_weight, W):
    n, d = x.shape
    d_out = W.shape[1]
    e = edge_weight.shape[0]

    dst = edge_index[0].astype(jnp.int32)
    src = edge_index[1].astype(jnp.int32)
    ew = edge_weight.astype(jnp.float32)

    per_round = CHUNK * NW
    k_chunks = -(-e // per_round)
    e_pad = k_chunks * per_round
    if e_pad != e:
        pad = e_pad - e
        src = jnp.concatenate([src, jnp.zeros((pad,), jnp.int32)])
        dst = jnp.concatenate([dst, jnp.zeros((pad,), jnp.int32)])
        ew = jnp.concatenate([ew, jnp.zeros((pad,), jnp.float32)])

    n_pad = -(-n // (8 * NS)) * (8 * NS)
    x_p = jnp.pad(x, ((0, n_pad - n), (0, 0))) if n_pad != n else x

    part = _sc_aggregate(x_p, src, dst, ew, n_pad, d, k_chunks)
    out = _tc_project(part, W, n_pad, d, d_out)
    return out[:n]
